# baseline (speedup 1.0000x reference)
import numpy as np

import jax
import jax.numpy as jnp
from jax.experimental import pallas as pl
from jax.experimental.pallas import tpu as pltpu

_ANCHOR_SIZES = [64.0, 128.0, 256.0]
_ANCHOR_RATIOS = [float(np.sqrt(r)) for r in [0.5, 1.0, 2.0]]
_ANCHORS = np.array(
    [[s * r, s / r] for s in _ANCHOR_SIZES for r in _ANCHOR_RATIOS],
    dtype=np.float32,
)

_PRE_NMS = 6000
_POST_NMS = 300
_IOU_THR = 0.7
_NEG = -1e9
_NEGT = -1e8
_BIGF = 1e9
_N = 9216
_ROWS = 72
_LANES = 128
_B = 4
_SL = 128


def _iota2(shape, dim):
    return jax.lax.broadcasted_iota(jnp.int32, shape, dim)


def _redmax(x):
    return jnp.max(jnp.max(x, axis=1, keepdims=True), axis=2, keepdims=True)


def _redmin(x):
    return jnp.min(jnp.min(x, axis=1, keepdims=True), axis=2, keepdims=True)


def _redsum(x):
    return jnp.sum(jnp.sum(x, axis=1, keepdims=True), axis=2, keepdims=True)


def _lmax(x):
    return jnp.max(x, axis=1, keepdims=True)


def _lmin(x):
    return jnp.min(x, axis=1, keepdims=True)


_DOT = dict(precision=jax.lax.Precision.HIGHEST,
            preferred_element_type=jnp.float32)


def _body(s_ref, tx_ref, ty_ref, tw_ref, th_ref, out_ref, act_ref, pend_ref):
    s = s_ref[...]
    shape3 = s.shape

    flat = _iota2((_ROWS, _LANES), 0) * _LANES + _iota2((_ROWS, _LANES), 1)
    a_idx = flat >> 10
    hw = flat & 1023
    hh = (hw >> 5).astype(jnp.float32)
    ww = (hw & 31).astype(jnp.float32)
    flat_f = flat.astype(jnp.float32)

    wa = jnp.zeros((_ROWS, _LANES), jnp.float32)
    ha = jnp.zeros((_ROWS, _LANES), jnp.float32)
    for k in range(9):
        sel = a_idx == k
        wa = jnp.where(sel, jnp.float32(_ANCHORS[k, 0]), wa)
        ha = jnp.where(sel, jnp.float32(_ANCHORS[k, 1]), ha)

    px = (ww + 0.5) * 16.0
    py = (hh + 0.5) * 16.0
    ax1 = px - wa / 2.0
    ay1 = py - ha / 2.0
    cx = ax1 + 0.5 * wa
    cy = ay1 + 0.5 * ha

    ncx = cx + tx_ref[...] * wa
    ncy = cy + ty_ref[...] * ha
    nw = wa * jnp.exp(tw_ref[...])
    nh = ha * jnp.exp(th_ref[...])
    bx1 = jnp.clip(ncx - 0.5 * nw, 0.0, 511.0)
    by1 = jnp.clip(ncy - 0.5 * nh, 0.0, 511.0)
    bx2 = jnp.clip(ncx + 0.5 * nw, 0.0, 511.0)
    by2 = jnp.clip(ncy + 0.5 * nh, 0.0, 511.0)
    area = jnp.maximum(bx2 - bx1, 0.0) * jnp.maximum(by2 - by1, 0.0)

    s_bits = jax.lax.bitcast_convert_type(s, jnp.int32)

    def bs6000(_, carry):
        lo, hi = carry
        mid = (lo + hi) >> 1
        cnt = _redsum(jnp.where(s_bits > mid, 1.0, 0.0))
        pred = cnt < float(_PRE_NMS)
        return jnp.where(pred, lo, mid + 1), jnp.where(pred, mid, hi)

    lo0 = jnp.zeros((_B, 1, 1), jnp.int32)
    hi0 = jnp.full((_B, 1, 1), 0x3F800000, jnp.int32)
    lo_f, _ = jax.lax.fori_loop(0, 31, bs6000, (lo0, hi0))
    thr = jax.lax.bitcast_convert_type(lo_f, jnp.float32)

    lane_lt = jnp.where(
        _iota2((_LANES, _LANES), 0) < _iota2((_LANES, _LANES), 1), 1.0, 0.0)
    p_i = _iota2((_B * _ROWS, _B * _ROWS), 0)
    q_i = _iota2((_B * _ROWS, _B * _ROWS), 1)
    row_lt = jnp.where(((p_i // _ROWS) == (q_i // _ROWS)) & (q_i < p_i),
                       1.0, 0.0)

    def eprefix(mask):
        mf = jnp.where(mask, 1.0, 0.0).reshape(_B * _ROWS, _LANES)
        in_row = jax.lax.dot(mf, lane_lt, **_DOT)
        rowsum = jnp.sum(mf, axis=1, keepdims=True)
        row_off = jax.lax.dot(row_lt, rowsum, **_DOT)
        return (in_row + row_off).reshape(shape3)

    gt = s > thr
    eq = s == thr
    cg = _redsum(jnp.where(gt, 1.0, 0.0))
    member = gt | (eq & (eprefix(eq) < float(_PRE_NMS) - cg))
    act_ref[...] = jnp.where(member, s, _NEG)

    m0 = _redmax(s)
    i0 = _redmin(jnp.where(s == m0, flat_f, _BIGF))
    oh0 = flat_f == i0
    st0 = jnp.concatenate(
        [jnp.where(oh0, c, _NEG) for c in (bx1, by1, bx2, by2)], axis=0)
    w0 = _redmax(st0)
    i0c = [w0[k * _B:(k + 1) * _B].reshape(_B, 1) for k in range(4)]
    i0f = i0.reshape(_B, 1)

    lane_i = _iota2((_B, _SL), 1).astype(jnp.float32)

    def refill(t, last_t, exh):
        p = t - last_t

        def appl(j, acc):
            row = pend_ref[j]
            wx1 = row[:, 0:1].reshape(_B, 1, 1)
            wy1 = row[:, 1:2].reshape(_B, 1, 1)
            wx2 = row[:, 2:3].reshape(_B, 1, 1)
            wy2 = row[:, 3:4].reshape(_B, 1, 1)
            wif = row[:, 4:5].reshape(_B, 1, 1)
            xx1 = jnp.maximum(wx1, bx1)
            yy1 = jnp.maximum(wy1, by1)
            xx2 = jnp.minimum(wx2, bx2)
            yy2 = jnp.minimum(wy2, by2)
            inter = (jnp.maximum(xx2 - xx1, 0.0)
                     * jnp.maximum(yy2 - yy1, 0.0))
            wa1 = (jnp.maximum(wx2 - wx1, 0.0)
                   * jnp.maximum(wy2 - wy1, 0.0))
            iou = inter / (wa1 + area - inter + 1e-8)
            return jnp.where((iou > _IOU_THR) | (flat_f == wif), 1.0, acc)

        supp = jax.lax.fori_loop(
            0, p, appl, jnp.zeros(shape3, jnp.float32))
        act = jnp.where(supp > 0.5, _NEG, act_ref[...])
        act_ref[...] = act

        a_bits = jax.lax.bitcast_convert_type(act, jnp.int32)

        def bs128(_, carry):
            lo, hi = carry
            mid = (lo + hi) >> 1
            cnt = _redsum(jnp.where(a_bits > mid, 1.0, 0.0))
            pred = cnt < float(_SL + 1)
            return jnp.where(pred, lo, mid + 1), jnp.where(pred, mid, hi)

        lo1, _ = jax.lax.fori_loop(
            0, 31, bs128,
            (jnp.zeros((_B, 1, 1), jnp.int32),
             jnp.full((_B, 1, 1), 0x3F800000, jnp.int32)))
        tau = jax.lax.bitcast_convert_type(lo1, jnp.float32)

        alive_any = _redmax(act) > _NEGT
        new_exh = jnp.where(alive_any, 0.0, 1.0).reshape(_B, 1)

        smask = jnp.where(act > tau, 1.0, 0.0)
        cgr = _redsum(smask)
        mm = _redmax(act)
        wi1 = _redmin(jnp.where(act == mm, flat_f, _BIGF))
        fb = (cgr < 0.5) & alive_any
        smask = jnp.where(fb, jnp.where(flat_f == wi1, 1.0, 0.0), smask)

        slot = eprefix(smask > 0.5)
        slcnt = _redsum(smask).reshape(_B, 1)

        slot_eff = jnp.where(smask > 0.5, slot, _BIGF).reshape(_B, 1, _N)
        at = jnp.where(
            slot_eff == _iota2((1, _SL, 1), 1).astype(jnp.float32),
            1.0, 0.0)
        lhs = jnp.concatenate(
            [v.reshape(_B, 1, _N)
             for v in (act, flat_f + jnp.zeros_like(act), bx1, by1,
                       bx2, by2)], axis=1)
        g = jax.lax.dot_general(
            lhs, at, (((2,), (2,)), ((0,), (0,))), **_DOT)
        fill = lane_i < slcnt
        n_score = jnp.where(fill, g[:, 0, :], _NEG)
        n_idx = g[:, 1, :]
        n_c = [g[:, 2 + k, :] for k in range(4)]
        return (n_score, n_idx, n_c[0], n_c[1], n_c[2], n_c[3],
                new_exh, t)

    def step(t, carry):
        sls, sli, slx1, sly1, slx2, sly2, exh, last_t = carry

        ms0 = _lmax(sls)
        need = jnp.max(jnp.where((ms0 < _NEGT) & (exh < 0.5), 1.0, 0.0))

        sls, sli, slx1, sly1, slx2, sly2, exh, last_t = jax.lax.cond(
            need > 0.5,
            lambda: refill(t, last_t, exh),
            lambda: (sls, sli, slx1, sly1, slx2, sly2, exh, last_t))

        ms = _lmax(sls)
        alive = ms > _NEGT
        wi = _lmin(jnp.where(sls == ms, sli, _BIGF))
        wif = jnp.where(alive, wi, i0f)
        oh = sli == wif

        st = jnp.concatenate(
            [jnp.where(oh, c, _NEG)
             for c in (slx1, sly1, slx2, sly2)], axis=0)
        w4 = jnp.max(st, axis=1, keepdims=True)
        wx1 = jnp.where(alive, w4[0:_B], i0c[0])
        wy1 = jnp.where(alive, w4[_B:2 * _B], i0c[1])
        wx2 = jnp.where(alive, w4[2 * _B:3 * _B], i0c[2])
        wy2 = jnp.where(alive, w4[3 * _B:], i0c[3])

        xx1 = jnp.maximum(wx1, slx1)
        yy1 = jnp.maximum(wy1, sly1)
        xx2 = jnp.minimum(wx2, slx2)
        yy2 = jnp.minimum(wy2, sly2)
        inter = jnp.maximum(xx2 - xx1, 0.0) * jnp.maximum(yy2 - yy1, 0.0)
        wa1 = jnp.maximum(wx2 - wx1, 0.0) * jnp.maximum(wy2 - wy1, 0.0)
        sarea = (jnp.maximum(slx2 - slx1, 0.0)
                 * jnp.maximum(sly2 - sly1, 0.0))
        iou = inter / (wa1 + sarea - inter + 1e-8)
        sls = jnp.where((iou > _IOU_THR) | oh, _NEG, sls)

        row4 = jnp.concatenate([wx1, wy1, wx2, wy2], axis=1)
        out_ref[t] = row4
        pend_ref[t - last_t] = jnp.concatenate(
            [row4, wif, jnp.zeros((_B, 3), jnp.float32)], axis=1)

        return (sls, sli, slx1, sly1, slx2, sly2, exh, last_t)

    zneg = jnp.full((_B, _SL), _NEG, jnp.float32)
    zf = jnp.zeros((_B, _SL), jnp.float32)
    jax.lax.fori_loop(
        0, _POST_NMS, step,
        (zneg, zf, zf, zf, zf, zf, jnp.zeros((_B, 1), jnp.float32),
         jnp.int32(0)))


def kernel(rpn_scores, rpn_deltas, input_image):
    del input_image
    s = rpn_scores.reshape(_B, _ROWS, _LANES)
    tx = rpn_deltas[:, 0::4].reshape(_B, _ROWS, _LANES)
    ty = rpn_deltas[:, 1::4].reshape(_B, _ROWS, _LANES)
    tw = rpn_deltas[:, 2::4].reshape(_B, _ROWS, _LANES)
    th = rpn_deltas[:, 3::4].reshape(_B, _ROWS, _LANES)
    out = pl.pallas_call(
        _body,
        out_shape=jax.ShapeDtypeStruct((_POST_NMS, _B, 4), jnp.float32),
        scratch_shapes=[
            pltpu.VMEM((_B, _ROWS, _LANES), jnp.float32),
            pltpu.VMEM((_POST_NMS + 4, _B, 8), jnp.float32),
        ],
    )(s, tx, ty, tw, th)
    return out.transpose(1, 0, 2)


# speedup vs baseline: 2.1881x; 2.1881x over previous
import numpy as np

import jax
import jax.numpy as jnp
from jax.experimental import pallas as pl

_ANCHOR_SIZES = [64.0, 128.0, 256.0]
_ANCHOR_RATIOS = [float(np.sqrt(r)) for r in [0.5, 1.0, 2.0]]
_ANCHORS = np.array(
    [[s * r, s / r] for s in _ANCHOR_SIZES for r in _ANCHOR_RATIOS],
    dtype=np.float32,
)

_PRE_NMS = 6000
_POST_NMS = 300
_IOU_THR = 0.7
_NEG = -1e9
_BIG_IDX = 1 << 30
_ROWS = 72
_LANES = 128
_B = 4


def _iota2(shape, dim):
    return jax.lax.broadcasted_iota(jnp.int32, shape, dim)


def _redmax(x):
    return jnp.max(jnp.max(x, axis=1, keepdims=True), axis=2, keepdims=True)


def _redmin(x):
    return jnp.min(jnp.min(x, axis=1, keepdims=True), axis=2, keepdims=True)


def _redsum(x):
    return jnp.sum(jnp.sum(x, axis=1, keepdims=True), axis=2, keepdims=True)


def _nms_body(s_ref, tx_ref, ty_ref, tw_ref, th_ref, out_ref):
    s = s_ref[...]
    shape3 = s.shape

    flat = _iota2((_ROWS, _LANES), 0) * _LANES + _iota2((_ROWS, _LANES), 1)
    a_idx = flat >> 10
    hw = flat & 1023
    hh = (hw >> 5).astype(jnp.float32)
    ww = (hw & 31).astype(jnp.float32)

    wa = jnp.zeros((_ROWS, _LANES), jnp.float32)
    ha = jnp.zeros((_ROWS, _LANES), jnp.float32)
    for k in range(9):
        sel = a_idx == k
        wa = jnp.where(sel, jnp.float32(_ANCHORS[k, 0]), wa)
        ha = jnp.where(sel, jnp.float32(_ANCHORS[k, 1]), ha)

    px = (ww + 0.5) * 16.0
    py = (hh + 0.5) * 16.0
    ax1 = px - wa / 2.0
    ay1 = py - ha / 2.0
    cx = ax1 + 0.5 * wa
    cy = ay1 + 0.5 * ha

    ncx = cx + tx_ref[...] * wa
    ncy = cy + ty_ref[...] * ha
    nw = wa * jnp.exp(tw_ref[...])
    nh = ha * jnp.exp(th_ref[...])
    bx1 = jnp.clip(ncx - 0.5 * nw, 0.0, 511.0)
    by1 = jnp.clip(ncy - 0.5 * nh, 0.0, 511.0)
    bx2 = jnp.clip(ncx + 0.5 * nw, 0.0, 511.0)
    by2 = jnp.clip(ncy + 0.5 * nh, 0.0, 511.0)
    area = jnp.maximum(bx2 - bx1, 0.0) * jnp.maximum(by2 - by1, 0.0)

    s_bits = jax.lax.bitcast_convert_type(s, jnp.int32)

    def bs_step(_, carry):
        lo, hi = carry
        mid = (lo + hi) >> 1
        cnt = _redsum(jnp.where(s_bits > mid, 1.0, 0.0))
        pred = cnt < float(_PRE_NMS)
        lo2 = jnp.where(pred, lo, mid + 1)
        hi2 = jnp.where(pred, mid, hi)
        return lo2, hi2

    lo0 = jnp.zeros((_B, 1, 1), jnp.int32)
    hi0 = jnp.full((_B, 1, 1), 0x3F800000, jnp.int32)
    lo_f, _ = jax.lax.fori_loop(0, 31, bs_step, (lo0, hi0))
    thr = jax.lax.bitcast_convert_type(lo_f, jnp.float32)

    gt = s > thr
    eq = s == thr
    cg = _redsum(jnp.where(gt, 1.0, 0.0))
    r_adm = float(_PRE_NMS) - cg

    eqf = jnp.where(eq, 1.0, 0.0).reshape(_B * _ROWS, _LANES)
    lane_lt = jnp.where(
        _iota2((_LANES, _LANES), 0) < _iota2((_LANES, _LANES), 1), 1.0, 0.0)
    in_row = jax.lax.dot(eqf, lane_lt,
                         precision=jax.lax.Precision.HIGHEST,
                         preferred_element_type=jnp.float32)
    rowsum = jnp.sum(eqf, axis=1, keepdims=True)
    p = _iota2((_B * _ROWS, _B * _ROWS), 0)
    q = _iota2((_B * _ROWS, _B * _ROWS), 1)
    row_lt = jnp.where(((p // _ROWS) == (q // _ROWS)) & (q < p), 1.0, 0.0)
    row_off = jax.lax.dot(row_lt, rowsum,
                          precision=jax.lax.Precision.HIGHEST,
                          preferred_element_type=jnp.float32)
    prefix = (in_row + row_off).reshape(shape3)

    member = gt | (eq & (prefix < r_adm))
    active0 = jnp.where(member, s, _NEG)

    flat_f = flat.astype(jnp.float32)
    m0 = _redmax(s)
    i0 = _redmin(jnp.where(s == m0, flat_f, jnp.float32(_BIG_IDX)))

    def step(t, active):
        m = _redmax(active)
        wi_raw = _redmin(jnp.where(active == m, flat_f, jnp.float32(_BIG_IDX)))
        wi = jnp.where(m > _NEG, wi_raw, i0)
        onehot = flat_f == wi
        stacked = jnp.concatenate(
            [jnp.where(onehot, c, _NEG) for c in (bx1, by1, bx2, by2)],
            axis=0)
        w4 = _redmax(stacked)
        wx1 = w4[0:_B]
        wy1 = w4[_B:2 * _B]
        wx2 = w4[2 * _B:3 * _B]
        wy2 = w4[3 * _B:]

        xx1 = jnp.maximum(wx1, bx1)
        yy1 = jnp.maximum(wy1, by1)
        xx2 = jnp.minimum(wx2, bx2)
        yy2 = jnp.minimum(wy2, by2)
        inter = jnp.maximum(xx2 - xx1, 0.0) * jnp.maximum(yy2 - yy1, 0.0)
        wa1 = jnp.maximum(wx2 - wx1, 0.0) * jnp.maximum(wy2 - wy1, 0.0)
        iou = inter / (wa1 + area - inter + 1e-8)

        new_active = jnp.where((iou > _IOU_THR) | onehot, _NEG, active)

        row = jnp.concatenate([wx1, wy1, wx2, wy2], axis=2)
        out_ref[t] = row.reshape(_B, 4)
        return new_active

    jax.lax.fori_loop(0, _POST_NMS, step, active0, unroll=2)


def kernel(rpn_scores, rpn_deltas, input_image):
    del input_image
    s = rpn_scores.reshape(_B, _ROWS, _LANES)
    tx = rpn_deltas[:, 0::4].reshape(_B, _ROWS, _LANES)
    ty = rpn_deltas[:, 1::4].reshape(_B, _ROWS, _LANES)
    tw = rpn_deltas[:, 2::4].reshape(_B, _ROWS, _LANES)
    th = rpn_deltas[:, 3::4].reshape(_B, _ROWS, _LANES)
    out = pl.pallas_call(
        _nms_body,
        out_shape=jax.ShapeDtypeStruct((_POST_NMS, _B, 4), jnp.float32),
    )(s, tx, ty, tw, th)
    return out.transpose(1, 0, 2)


# speedup vs baseline: 2.2852x; 1.0444x over previous
import numpy as np

import jax
import jax.numpy as jnp
from jax.experimental import pallas as pl

_ANCHOR_SIZES = [64.0, 128.0, 256.0]
_ANCHOR_RATIOS = [float(np.sqrt(r)) for r in [0.5, 1.0, 2.0]]
_ANCHORS = np.array(
    [[s * r, s / r] for s in _ANCHOR_SIZES for r in _ANCHOR_RATIOS],
    dtype=np.float32,
)

_PRE_NMS = 6000
_POST_NMS = 300
_IOU_THR = 0.7
_NEG = -1e9
_BIG_IDX = 1 << 30
_ROWS = 72
_LANES = 128
_B = 4


def _iota2(shape, dim):
    return jax.lax.broadcasted_iota(jnp.int32, shape, dim)


def _redmax(x):
    return jnp.max(jnp.max(x, axis=1, keepdims=True), axis=2, keepdims=True)


def _redmin(x):
    return jnp.min(jnp.min(x, axis=1, keepdims=True), axis=2, keepdims=True)


def _redsum(x):
    return jnp.sum(jnp.sum(x, axis=1, keepdims=True), axis=2, keepdims=True)


def _nms_body(s_ref, tx_ref, ty_ref, tw_ref, th_ref, out_ref):
    s = s_ref[...]
    shape3 = s.shape

    flat = _iota2((_ROWS, _LANES), 0) * _LANES + _iota2((_ROWS, _LANES), 1)
    a_idx = flat >> 10
    hw = flat & 1023
    hh = (hw >> 5).astype(jnp.float32)
    ww = (hw & 31).astype(jnp.float32)

    wa = jnp.zeros((_ROWS, _LANES), jnp.float32)
    ha = jnp.zeros((_ROWS, _LANES), jnp.float32)
    for k in range(9):
        sel = a_idx == k
        wa = jnp.where(sel, jnp.float32(_ANCHORS[k, 0]), wa)
        ha = jnp.where(sel, jnp.float32(_ANCHORS[k, 1]), ha)

    px = (ww + 0.5) * 16.0
    py = (hh + 0.5) * 16.0
    ax1 = px - wa / 2.0
    ay1 = py - ha / 2.0
    cx = ax1 + 0.5 * wa
    cy = ay1 + 0.5 * ha

    ncx = cx + tx_ref[...] * wa
    ncy = cy + ty_ref[...] * ha
    nw = wa * jnp.exp(tw_ref[...])
    nh = ha * jnp.exp(th_ref[...])
    bx1 = jnp.clip(ncx - 0.5 * nw, 0.0, 511.0)
    by1 = jnp.clip(ncy - 0.5 * nh, 0.0, 511.0)
    bx2 = jnp.clip(ncx + 0.5 * nw, 0.0, 511.0)
    by2 = jnp.clip(ncy + 0.5 * nh, 0.0, 511.0)
    area = jnp.maximum(bx2 - bx1, 0.0) * jnp.maximum(by2 - by1, 0.0)

    s_bits = jax.lax.bitcast_convert_type(s, jnp.int32)

    def bs_step(_, carry):
        lo, hi = carry
        mid = (lo + hi) >> 1
        cnt = _redsum(jnp.where(s_bits > mid, 1.0, 0.0))
        pred = cnt < float(_PRE_NMS)
        lo2 = jnp.where(pred, lo, mid + 1)
        hi2 = jnp.where(pred, mid, hi)
        return lo2, hi2

    lo0 = jnp.zeros((_B, 1, 1), jnp.int32)
    hi0 = jnp.full((_B, 1, 1), 0x3F800000, jnp.int32)
    lo_f, _ = jax.lax.fori_loop(0, 31, bs_step, (lo0, hi0))
    thr = jax.lax.bitcast_convert_type(lo_f, jnp.float32)

    gt = s > thr
    eq = s == thr
    cg = _redsum(jnp.where(gt, 1.0, 0.0))
    r_adm = float(_PRE_NMS) - cg

    eqf = jnp.where(eq, 1.0, 0.0).reshape(_B * _ROWS, _LANES)
    lane_lt = jnp.where(
        _iota2((_LANES, _LANES), 0) < _iota2((_LANES, _LANES), 1), 1.0, 0.0)
    in_row = jax.lax.dot(eqf, lane_lt,
                         precision=jax.lax.Precision.HIGHEST,
                         preferred_element_type=jnp.float32)
    rowsum = jnp.sum(eqf, axis=1, keepdims=True)
    p = _iota2((_B * _ROWS, _B * _ROWS), 0)
    q = _iota2((_B * _ROWS, _B * _ROWS), 1)
    row_lt = jnp.where(((p // _ROWS) == (q // _ROWS)) & (q < p), 1.0, 0.0)
    row_off = jax.lax.dot(row_lt, rowsum,
                          precision=jax.lax.Precision.HIGHEST,
                          preferred_element_type=jnp.float32)
    prefix = (in_row + row_off).reshape(shape3)

    member = gt | (eq & (prefix < r_adm))
    active0 = jnp.where(member, s, _NEG)

    flat_f = flat.astype(jnp.float32)
    m0 = _redmax(s)
    i0 = _redmin(jnp.where(s == m0, flat_f, jnp.float32(_BIG_IDX)))

    def step(t, active):
        m = _redmax(active)
        wi_raw = _redmin(jnp.where(active == m, flat_f, jnp.float32(_BIG_IDX)))
        wi = jnp.where(m > _NEG, wi_raw, i0)
        onehot = flat_f == wi
        stacked = jnp.concatenate(
            [jnp.where(onehot, c, _NEG) for c in (bx1, by1, bx2, by2)],
            axis=0)
        w4 = _redmax(stacked)
        wx1 = w4[0:_B]
        wy1 = w4[_B:2 * _B]
        wx2 = w4[2 * _B:3 * _B]
        wy2 = w4[3 * _B:]

        xx1 = jnp.maximum(wx1, bx1)
        yy1 = jnp.maximum(wy1, by1)
        xx2 = jnp.minimum(wx2, bx2)
        yy2 = jnp.minimum(wy2, by2)
        inter = jnp.maximum(xx2 - xx1, 0.0) * jnp.maximum(yy2 - yy1, 0.0)
        wa1 = jnp.maximum(wx2 - wx1, 0.0) * jnp.maximum(wy2 - wy1, 0.0)
        iou = inter / (wa1 + area - inter + 1e-8)

        new_active = jnp.where((iou > _IOU_THR) | onehot, _NEG, active)

        row = jnp.concatenate([wx1, wy1, wx2, wy2], axis=2)
        out_ref[t] = row.reshape(_B, 4)
        return new_active

    jax.lax.fori_loop(0, _POST_NMS, step, active0, unroll=4)


def kernel(rpn_scores, rpn_deltas, input_image):
    del input_image
    s = rpn_scores.reshape(_B, _ROWS, _LANES)
    tx = rpn_deltas[:, 0::4].reshape(_B, _ROWS, _LANES)
    ty = rpn_deltas[:, 1::4].reshape(_B, _ROWS, _LANES)
    tw = rpn_deltas[:, 2::4].reshape(_B, _ROWS, _LANES)
    th = rpn_deltas[:, 3::4].reshape(_B, _ROWS, _LANES)
    out = pl.pallas_call(
        _nms_body,
        out_shape=jax.ShapeDtypeStruct((_POST_NMS, _B, 4), jnp.float32),
    )(s, tx, ty, tw, th)
    return out.transpose(1, 0, 2)
